# baseline (speedup 1.0000x reference)
import functools

import jax
import jax.numpy as jnp
from jax import lax
from jax.experimental import pallas as pl
from jax.experimental.pallas import tpu as pltpu
from jax.experimental.pallas import tpu_sc as plsc

N = 10000
E = 320000
H = 128
HW = H // 2
NC, NS, LANES = 2, 16, 16
NW = NC * NS
EPW = E // NW
GC = 80
NGC = EPW // GC
SC2 = 2000
NSC2 = EPW // SC2
G16 = SC2 // LANES

_SC_MESH = plsc.VectorSubcoreMesh(core_axis_name="c", subcore_axis_name="s")


def _pack_bf16_pairs(p):
    b = lax.bitcast_convert_type(p, jnp.int32)
    r = (b + 0x7FFF + ((b >> 16) & 1)) >> 16
    lo = r[:, :HW] & 0xFFFF
    hi = r[:, HW:] << 16
    return hi | lo


def _unpack_bf16_pairs(g):
    f_lo = lax.bitcast_convert_type(g << 16, jnp.float32)
    f_hi = lax.bitcast_convert_type((g >> 16) << 16, jnp.float32)
    return jnp.concatenate([f_lo, f_hi], axis=1)


def _precompute_body(cond_ref, wc1_ref, bc1_ref, be2_ref, t_ref,
                     psp_ref, pdp_ref, bias_ref):
    cnd = cond_ref[...]
    wa = wc1_ref[0:128, :]
    wts = wc1_ref[127:128, :]
    wb = wc1_ref[128:256, :]
    wtd = wc1_ref[255:256, :]
    wc = wc1_ref[256:288, :]
    ps = jnp.dot(cnd, wa, preferred_element_type=jnp.float32, precision=lax.Precision.HIGHEST)
    pd = jnp.dot(cnd, wb, preferred_element_type=jnp.float32, precision=lax.Precision.HIGHEST)
    psp_ref[...] = _pack_bf16_pairs(ps)
    pdp_ref[...] = _pack_bf16_pairs(pd)
    bias_ref[...] = (bc1_ref[...]
                     + jnp.dot(be2_ref[...], wc,
                               preferred_element_type=jnp.float32, precision=lax.Precision.HIGHEST)
                     + t_ref[...] * (wts + wtd))


_precompute = pl.pallas_call(
    _precompute_body,
    out_shape=[jax.ShapeDtypeStruct((N, HW), jnp.int32),
               jax.ShapeDtypeStruct((N, HW), jnp.int32),
               jax.ShapeDtypeStruct((1, H), jnp.float32)],
)


@functools.partial(
    pl.kernel,
    out_type=[jax.ShapeDtypeStruct((E, HW), jnp.int32),
              jax.ShapeDtypeStruct((E, HW), jnp.int32)],
    mesh=_SC_MESH,
    scratch_types=[
        pltpu.VMEM((GC,), jnp.int32), pltpu.VMEM((GC,), jnp.int32),
        pltpu.VMEM((GC,), jnp.int32), pltpu.VMEM((GC,), jnp.int32),
        pltpu.VMEM((GC, HW), jnp.int32), pltpu.VMEM((GC, HW), jnp.int32),
        pltpu.VMEM((GC, HW), jnp.int32), pltpu.VMEM((GC, HW), jnp.int32),
        pltpu.SemaphoreType.DMA, pltpu.SemaphoreType.DMA,
        pltpu.SemaphoreType.DMA, pltpu.SemaphoreType.DMA,
        pltpu.SemaphoreType.DMA, pltpu.SemaphoreType.DMA,
        pltpu.SemaphoreType.DMA, pltpu.SemaphoreType.DMA,
    ],
    compiler_params=pltpu.CompilerParams(use_tc_tiling_on_sc=False),
)
def _gather_kernel(psp_hbm, pdp_hbm, src_hbm, dst_hbm, gs_hbm, gd_hbm,
                   is0, is1, id0, id1, bs0, bs1, bd0, bd1,
                   sis0, sis1, sid0, sid1, sgs0, sgs1, sgd0, sgd1):
    wid = lax.axis_index("s") * NC + lax.axis_index("c")
    base = wid * EPW
    IS, ID = (is0, is1), (id0, id1)
    BS, BD = (bs0, bs1), (bd0, bd1)
    SIS, SID = (sis0, sis1), (sid0, sid1)
    SGS, SGD = (sgs0, sgs1), (sgd0, sgd1)

    def idx_cp(b, c):
        off = base + c * GC
        return (pltpu.make_async_copy(src_hbm.at[pl.ds(off, GC)], IS[b], SIS[b]),
                pltpu.make_async_copy(dst_hbm.at[pl.ds(off, GC)], ID[b], SID[b]))

    def issue_idx(b, c):
        c1, c2 = idx_cp(b, c)
        c1.start()
        c2.start()

    def wait_idx(b):
        c1, c2 = idx_cp(b, 0)
        c1.wait()
        c2.wait()

    def gath(b):
        return (pltpu.make_async_copy(psp_hbm.at[IS[b]], BS[b], SGS[b]),
                pltpu.make_async_copy(pdp_hbm.at[ID[b]], BD[b], SGD[b]))

    def issue_g(b):
        g1, g2 = gath(b)
        g1.start()
        g2.start()

    def wait_g(b):
        g1, g2 = gath(b)
        g1.wait()
        g2.wait()

    def write(b, c):
        off = base + c * GC
        pltpu.sync_copy(BS[b], gs_hbm.at[pl.ds(off, GC)])
        pltpu.sync_copy(BD[b], gd_hbm.at[pl.ds(off, GC)])

    issue_idx(0, 0)
    wait_idx(0)
    issue_g(0)
    issue_idx(1, 1)

    def pair(j, carry):
        c = 2 * j + 1
        wait_idx(1)
        issue_g(1)
        wait_g(0)
        write(0, c - 1)
        issue_idx(0, c + 1)
        wait_idx(0)
        issue_g(0)
        wait_g(1)
        write(1, c)

        @pl.when(j < (NGC // 2) - 1)
        def _():
            issue_idx(1, c + 2)

        return carry

    lax.fori_loop(0, NGC // 2, pair, 0)
    wait_g(0)
    write(0, NGC - 1)


BE = 4000


def _edge_body(gsp_ref, gdp_ref, d_ref, we1_ref, be1_ref, we2_ref, wc1c_ref,
               wc2_ref, bias_ref, c_ref):
    d = d_ref[...]
    a = d * we1_ref[...] + be1_ref[...]
    sa = a * jax.nn.sigmoid(a)
    wcomb = jnp.dot(we2_ref[...], wc1c_ref[...],
                    preferred_element_type=jnp.float32, precision=lax.Precision.HIGHEST)
    q = jnp.dot(sa, wcomb, preferred_element_type=jnp.float32, precision=lax.Precision.HIGHEST)
    gs = _unpack_bf16_pairs(gsp_ref[...])
    gd = _unpack_bf16_pairs(gdp_ref[...])
    u = gs + gd + q + bias_ref[...]
    su = u * jax.nn.sigmoid(u)
    c_ref[...] = jnp.dot(su, wc2_ref[...], preferred_element_type=jnp.float32, precision=lax.Precision.HIGHEST)


_edge_epilogue = pl.pallas_call(
    _edge_body,
    grid=(E // BE,),
    in_specs=[
        pl.BlockSpec((BE, HW), lambda i: (i, 0)),
        pl.BlockSpec((BE, HW), lambda i: (i, 0)),
        pl.BlockSpec((BE, 1), lambda i: (i, 0)),
        pl.BlockSpec((1, 32), lambda i: (0, 0)),
        pl.BlockSpec((1, 32), lambda i: (0, 0)),
        pl.BlockSpec((32, 32), lambda i: (0, 0)),
        pl.BlockSpec((32, H), lambda i: (0, 0)),
        pl.BlockSpec((H, 1), lambda i: (0, 0)),
        pl.BlockSpec((1, H), lambda i: (0, 0)),
    ],
    out_specs=pl.BlockSpec((BE, 1), lambda i: (i, 0)),
    out_shape=jax.ShapeDtypeStruct((E, 1), jnp.float32),
)


@functools.partial(
    pl.kernel,
    out_type=jax.ShapeDtypeStruct((NW * 3 * N,), jnp.float32),
    mesh=_SC_MESH,
    scratch_types=[
        pltpu.VMEM((N,), jnp.float32),
        pltpu.VMEM((N,), jnp.float32),
        pltpu.VMEM((N,), jnp.float32),
        pltpu.VMEM((N,), jnp.float32),
        pltpu.VMEM((N,), jnp.float32),
        pltpu.VMEM((N,), jnp.float32),
        pltpu.VMEM((SC2,), jnp.int32),
        pltpu.VMEM((SC2,), jnp.int32),
        pltpu.VMEM((SC2,), jnp.float32),
    ],
    compiler_params=pltpu.CompilerParams(needs_layout_passes=False),
)
def _scatter_kernel(xt_hbm, src_hbm, dst_hbm, c_hbm, out_hbm,
                    xv, yv, zv, ax, ay, az, sv, dv, cv):
    wid = lax.axis_index("s") * NC + lax.axis_index("c")
    base = wid * EPW
    pltpu.sync_copy(xt_hbm.at[pl.ds(0, N)], xv)
    pltpu.sync_copy(xt_hbm.at[pl.ds(N, N)], yv)
    pltpu.sync_copy(xt_hbm.at[pl.ds(2 * N, N)], zv)
    zeros = jnp.zeros((LANES,), jnp.float32)

    def zbody(i, carry):
        ax[pl.ds(i * LANES, LANES)] = zeros
        ay[pl.ds(i * LANES, LANES)] = zeros
        az[pl.ds(i * LANES, LANES)] = zeros
        return carry

    lax.fori_loop(0, N // LANES, zbody, 0)

    def chunk(ci, carry):
        off = base + ci * SC2
        pltpu.sync_copy(src_hbm.at[pl.ds(off, SC2)], sv)
        pltpu.sync_copy(dst_hbm.at[pl.ds(off, SC2)], dv)
        pltpu.sync_copy(c_hbm.at[pl.ds(off, SC2)], cv)

        def grp(g, c2):
            s = sv[pl.ds(g * LANES, LANES)]
            dd = dv[pl.ds(g * LANES, LANES)]
            xs = plsc.load_gather(xv, [s])
            xd = plsc.load_gather(xv, [dd])
            ys = plsc.load_gather(yv, [s])
            yd = plsc.load_gather(yv, [dd])
            zs = plsc.load_gather(zv, [s])
            zd = plsc.load_gather(zv, [dd])
            dx = xs - xd
            dy = ys - yd
            dz = zs - zd
            n2 = jnp.maximum(dx * dx + dy * dy + dz * dz,
                             jnp.float32(1e-16))
            ib = plsc.bitcast(n2, jnp.int32)
            yb = jnp.int32(0x5F3759DF) - lax.shift_right_logical(ib, 1)
            yr = plsc.bitcast(yb, jnp.float32)
            yr = yr * (1.5 - 0.5 * n2 * yr * yr)
            yr = yr * (1.5 - 0.5 * n2 * yr * yr)
            yr = yr * (1.5 - 0.5 * n2 * yr * yr)
            cc = cv[pl.ds(g * LANES, LANES)]
            s_c = cc * yr
            plsc.addupdate_scatter(ax, [dd], s_c * dx)
            plsc.addupdate_scatter(ay, [dd], s_c * dy)
            plsc.addupdate_scatter(az, [dd], s_c * dz)
            return c2

        lax.fori_loop(0, G16, grp, 0)
        return carry

    lax.fori_loop(0, NSC2, chunk, 0)
    obase = wid * (3 * N)
    pltpu.sync_copy(ax, out_hbm.at[pl.ds(obase, N)])
    pltpu.sync_copy(ay, out_hbm.at[pl.ds(obase + N, N)])
    pltpu.sync_copy(az, out_hbm.at[pl.ds(obase + 2 * N, N)])


def _reduce_body(part_ref, xt_ref, o_ref):
    o_ref[...] = xt_ref[...] + jnp.sum(part_ref[...], axis=0)


_reduce = pl.pallas_call(
    _reduce_body,
    out_shape=jax.ShapeDtypeStruct((3, N), jnp.float32),
)


def kernel(x_t, cond, t, edge_index, edge_dist, Wn1, bn1, Wn2, bn2,
           Wc1, bc1, Wc2, We1, be1, We2, be2):
    B = x_t.shape[0]
    src = edge_index[0]
    dst = edge_index[1]
    x3n = x_t.reshape(N, 3).T
    tsc = jnp.full((1, 1), t, dtype=jnp.float32)

    cond0 = jnp.concatenate(
        [cond.reshape(N, H - 1), jnp.zeros((N, 1), jnp.float32)], axis=1)
    psp, pdp, bias = _precompute(cond0, Wc1,
                                 bc1.reshape(1, H), be2.reshape(1, 32), tsc)
    gsp, gdp = _gather_kernel(psp, pdp, src, dst)
    c = _edge_epilogue(gsp, gdp, edge_dist.reshape(E, 1), We1.reshape(1, 32),
                       be1.reshape(1, 32), We2, Wc1[256:288], Wc2,
                       bias)
    partials = _scatter_kernel(x3n.reshape(3 * N), src, dst, c.reshape(E))
    out3n = _reduce(partials.reshape(NW, 3, N), x3n)
    return out3n.T.reshape(B, N, 3)


# speedup vs baseline: 1.4445x; 1.4445x over previous
import functools

import jax
import jax.numpy as jnp
from jax import lax
from jax.experimental import pallas as pl
from jax.experimental.pallas import tpu as pltpu
from jax.experimental.pallas import tpu_sc as plsc

N = 10000
E = 320000
H = 128
HW = H // 2
NC, NS, LANES = 2, 16, 16
NW = NC * NS
EPW = E // NW
GC = 80
NGC = EPW // GC
SC2 = 2000
NSC2 = EPW // SC2
G16 = SC2 // LANES

_SC_MESH = plsc.VectorSubcoreMesh(core_axis_name="c", subcore_axis_name="s")


def _pack_bf16_pairs(p):
    b = lax.bitcast_convert_type(p, jnp.int32)
    r = (b + 0x7FFF + ((b >> 16) & 1)) >> 16
    lo = r[:, :HW] & 0xFFFF
    hi = r[:, HW:] << 16
    return hi | lo


def _unpack_bf16_pairs(g):
    f_lo = lax.bitcast_convert_type(g << 16, jnp.float32)
    f_hi = lax.bitcast_convert_type((g >> 16) << 16, jnp.float32)
    return jnp.concatenate([f_lo, f_hi], axis=1)


def _precompute_body(cond_ref, wc1_ref, bc1_ref, be2_ref, t_ref,
                     psp_ref, pdp_ref, bias_ref):
    cnd = cond_ref[...]
    wa = wc1_ref[0:128, :]
    wts = wc1_ref[127:128, :]
    wb = wc1_ref[128:256, :]
    wtd = wc1_ref[255:256, :]
    wc = wc1_ref[256:288, :]
    ps = jnp.dot(cnd, wa, preferred_element_type=jnp.float32, precision=lax.Precision.HIGHEST)
    pd = jnp.dot(cnd, wb, preferred_element_type=jnp.float32, precision=lax.Precision.HIGHEST)
    psp_ref[...] = _pack_bf16_pairs(ps)
    pdp_ref[...] = _pack_bf16_pairs(pd)
    bias_ref[...] = (bc1_ref[...]
                     + jnp.dot(be2_ref[...], wc,
                               preferred_element_type=jnp.float32, precision=lax.Precision.HIGHEST)
                     + t_ref[...] * (wts + wtd))


_precompute = pl.pallas_call(
    _precompute_body,
    out_shape=[jax.ShapeDtypeStruct((N, HW), jnp.int32),
               jax.ShapeDtypeStruct((N, HW), jnp.int32),
               jax.ShapeDtypeStruct((1, H), jnp.float32)],
)


@functools.partial(
    pl.kernel,
    out_type=[jax.ShapeDtypeStruct((E, HW), jnp.int32),
              jax.ShapeDtypeStruct((E, HW), jnp.int32)],
    mesh=_SC_MESH,
    scratch_types=[
        pltpu.VMEM((GC,), jnp.int32), pltpu.VMEM((GC,), jnp.int32),
        pltpu.VMEM((GC,), jnp.int32), pltpu.VMEM((GC,), jnp.int32),
        pltpu.VMEM((GC, HW), jnp.int32), pltpu.VMEM((GC, HW), jnp.int32),
        pltpu.VMEM((GC, HW), jnp.int32), pltpu.VMEM((GC, HW), jnp.int32),
        pltpu.SemaphoreType.DMA, pltpu.SemaphoreType.DMA,
        pltpu.SemaphoreType.DMA, pltpu.SemaphoreType.DMA,
        pltpu.SemaphoreType.DMA, pltpu.SemaphoreType.DMA,
        pltpu.SemaphoreType.DMA, pltpu.SemaphoreType.DMA,
    ],
    compiler_params=pltpu.CompilerParams(use_tc_tiling_on_sc=False),
)
def _gather_kernel(psp_hbm, pdp_hbm, src_hbm, dst_hbm, gs_hbm, gd_hbm,
                   is0, is1, id0, id1, bs0, bs1, bd0, bd1,
                   sis0, sis1, sid0, sid1, sgs0, sgs1, sgd0, sgd1):
    wid = lax.axis_index("s") * NC + lax.axis_index("c")
    base = wid * EPW
    IS, ID = (is0, is1), (id0, id1)
    BS, BD = (bs0, bs1), (bd0, bd1)
    SIS, SID = (sis0, sis1), (sid0, sid1)
    SGS, SGD = (sgs0, sgs1), (sgd0, sgd1)

    def idx_cp(b, c):
        off = base + c * GC
        return (pltpu.make_async_copy(src_hbm.at[pl.ds(off, GC)], IS[b], SIS[b]),
                pltpu.make_async_copy(dst_hbm.at[pl.ds(off, GC)], ID[b], SID[b]))

    def issue_idx(b, c):
        c1, c2 = idx_cp(b, c)
        c1.start()
        c2.start()

    def wait_idx(b):
        c1, c2 = idx_cp(b, 0)
        c1.wait()
        c2.wait()

    def gath(b):
        return (pltpu.make_async_copy(psp_hbm.at[IS[b]], BS[b], SGS[b]),
                pltpu.make_async_copy(pdp_hbm.at[ID[b]], BD[b], SGD[b]))

    def issue_g(b):
        g1, g2 = gath(b)
        g1.start()
        g2.start()

    def wait_g(b):
        g1, g2 = gath(b)
        g1.wait()
        g2.wait()

    def write(b, c):
        off = base + c * GC
        pltpu.sync_copy(BS[b], gs_hbm.at[pl.ds(off, GC)])
        pltpu.sync_copy(BD[b], gd_hbm.at[pl.ds(off, GC)])

    issue_idx(0, 0)
    wait_idx(0)
    issue_g(0)
    issue_idx(1, 1)

    def pair(j, carry):
        c = 2 * j + 1
        wait_idx(1)
        issue_g(1)
        wait_g(0)
        write(0, c - 1)
        issue_idx(0, c + 1)
        wait_idx(0)
        issue_g(0)
        wait_g(1)
        write(1, c)

        @pl.when(j < (NGC // 2) - 1)
        def _():
            issue_idx(1, c + 2)

        return carry

    lax.fori_loop(0, NGC // 2, pair, 0)
    wait_g(0)
    write(0, NGC - 1)


BEH = 2000
EH = E // 2


def _edge_body(gsp_ref, gdp_ref, d_ref, we1_ref, be1_ref, we2_ref, wc1c_ref,
               wc2_ref, bias_ref, c_ref):
    g_s = gsp_ref[...]
    g_d = gdp_ref[...]
    slo = lax.bitcast_convert_type(g_s << 16, jnp.float32)
    shi = lax.bitcast_convert_type((g_s >> 16) << 16, jnp.float32)
    dlo = lax.bitcast_convert_type(g_d << 16, jnp.float32)
    dhi = lax.bitcast_convert_type((g_d >> 16) << 16, jnp.float32)
    wcomb = jnp.dot(we2_ref[...], wc1c_ref[...],
                    preferred_element_type=jnp.float32,
                    precision=lax.Precision.HIGHEST)
    d2 = d_ref[...]
    bias = bias_ref[...]
    cs = []
    for p in (0, 1):
        gs = jnp.concatenate([slo[:, p * HW:p * HW + HW],
                              shi[:, p * HW:p * HW + HW]], axis=1)
        gd = jnp.concatenate([dlo[:, p * HW:p * HW + HW],
                              dhi[:, p * HW:p * HW + HW]], axis=1)
        a = d2[:, p:p + 1] * we1_ref[...] + be1_ref[...]
        sa = a * jax.nn.sigmoid(a)
        q = jnp.dot(sa, wcomb, preferred_element_type=jnp.float32,
                    precision=lax.Precision.HIGHEST)
        u = gs + gd + q + bias
        su = u * jax.nn.sigmoid(u)
        cs.append(jnp.dot(su, wc2_ref[...], preferred_element_type=jnp.float32,
                          precision=lax.Precision.HIGHEST))
    c_ref[...] = jnp.concatenate(cs, axis=1)


_edge_epilogue = pl.pallas_call(
    _edge_body,
    grid=(EH // BEH,),
    in_specs=[
        pl.BlockSpec((BEH, H), lambda i: (i, 0)),
        pl.BlockSpec((BEH, H), lambda i: (i, 0)),
        pl.BlockSpec((BEH, 2), lambda i: (i, 0)),
        pl.BlockSpec((1, 32), lambda i: (0, 0)),
        pl.BlockSpec((1, 32), lambda i: (0, 0)),
        pl.BlockSpec((32, 32), lambda i: (0, 0)),
        pl.BlockSpec((32, H), lambda i: (0, 0)),
        pl.BlockSpec((H, 1), lambda i: (0, 0)),
        pl.BlockSpec((1, H), lambda i: (0, 0)),
    ],
    out_specs=pl.BlockSpec((BEH, 2), lambda i: (i, 0)),
    out_shape=jax.ShapeDtypeStruct((EH, 2), jnp.float32),
)


@functools.partial(
    pl.kernel,
    out_type=jax.ShapeDtypeStruct((NW * 3 * N,), jnp.float32),
    mesh=_SC_MESH,
    scratch_types=[
        pltpu.VMEM((N,), jnp.float32),
        pltpu.VMEM((N,), jnp.float32),
        pltpu.VMEM((N,), jnp.float32),
        pltpu.VMEM((N,), jnp.float32),
        pltpu.VMEM((N,), jnp.float32),
        pltpu.VMEM((N,), jnp.float32),
        pltpu.VMEM((SC2,), jnp.int32),
        pltpu.VMEM((SC2,), jnp.int32),
        pltpu.VMEM((SC2,), jnp.float32),
    ],
    compiler_params=pltpu.CompilerParams(needs_layout_passes=False),
)
def _scatter_kernel(xt_hbm, src_hbm, dst_hbm, c_hbm, out_hbm,
                    xv, yv, zv, ax, ay, az, sv, dv, cv):
    wid = lax.axis_index("s") * NC + lax.axis_index("c")
    base = wid * EPW
    pltpu.sync_copy(xt_hbm.at[pl.ds(0, N)], xv)
    pltpu.sync_copy(xt_hbm.at[pl.ds(N, N)], yv)
    pltpu.sync_copy(xt_hbm.at[pl.ds(2 * N, N)], zv)
    zeros = jnp.zeros((LANES,), jnp.float32)

    def zbody(i, carry):
        ax[pl.ds(i * LANES, LANES)] = zeros
        ay[pl.ds(i * LANES, LANES)] = zeros
        az[pl.ds(i * LANES, LANES)] = zeros
        return carry

    lax.fori_loop(0, N // LANES, zbody, 0)

    def chunk(ci, carry):
        off = base + ci * SC2
        pltpu.sync_copy(src_hbm.at[pl.ds(off, SC2)], sv)
        pltpu.sync_copy(dst_hbm.at[pl.ds(off, SC2)], dv)
        pltpu.sync_copy(c_hbm.at[pl.ds(off, SC2)], cv)

        def grp(g, c2):
            s = sv[pl.ds(g * LANES, LANES)]
            dd = dv[pl.ds(g * LANES, LANES)]
            xs = plsc.load_gather(xv, [s])
            xd = plsc.load_gather(xv, [dd])
            ys = plsc.load_gather(yv, [s])
            yd = plsc.load_gather(yv, [dd])
            zs = plsc.load_gather(zv, [s])
            zd = plsc.load_gather(zv, [dd])
            dx = xs - xd
            dy = ys - yd
            dz = zs - zd
            n2 = jnp.maximum(dx * dx + dy * dy + dz * dz,
                             jnp.float32(1e-16))
            ib = plsc.bitcast(n2, jnp.int32)
            yb = jnp.int32(0x5F3759DF) - lax.shift_right_logical(ib, 1)
            yr = plsc.bitcast(yb, jnp.float32)
            yr = yr * (1.5 - 0.5 * n2 * yr * yr)
            yr = yr * (1.5 - 0.5 * n2 * yr * yr)
            yr = yr * (1.5 - 0.5 * n2 * yr * yr)
            cc = cv[pl.ds(g * LANES, LANES)]
            s_c = cc * yr
            plsc.addupdate_scatter(ax, [dd], s_c * dx)
            plsc.addupdate_scatter(ay, [dd], s_c * dy)
            plsc.addupdate_scatter(az, [dd], s_c * dz)
            return c2

        lax.fori_loop(0, G16, grp, 0)
        return carry

    lax.fori_loop(0, NSC2, chunk, 0)
    obase = wid * (3 * N)
    pltpu.sync_copy(ax, out_hbm.at[pl.ds(obase, N)])
    pltpu.sync_copy(ay, out_hbm.at[pl.ds(obase + N, N)])
    pltpu.sync_copy(az, out_hbm.at[pl.ds(obase + 2 * N, N)])


def _reduce_body(part_ref, xt_ref, o_ref):
    o_ref[...] = xt_ref[...] + jnp.sum(part_ref[...], axis=0)


_reduce = pl.pallas_call(
    _reduce_body,
    out_shape=jax.ShapeDtypeStruct((3, N), jnp.float32),
)


def kernel(x_t, cond, t, edge_index, edge_dist, Wn1, bn1, Wn2, bn2,
           Wc1, bc1, Wc2, We1, be1, We2, be2):
    B = x_t.shape[0]
    src = edge_index[0]
    dst = edge_index[1]
    x3n = x_t.reshape(N, 3).T
    tsc = jnp.full((1, 1), t, dtype=jnp.float32)

    cond0 = jnp.concatenate(
        [cond.reshape(N, H - 1), jnp.zeros((N, 1), jnp.float32)], axis=1)
    psp, pdp, bias = _precompute(cond0, Wc1,
                                 bc1.reshape(1, H), be2.reshape(1, 32), tsc)
    gsp, gdp = _gather_kernel(psp, pdp, src, dst)
    c = _edge_epilogue(gsp.reshape(EH, H), gdp.reshape(EH, H),
                       edge_dist.reshape(EH, 2), We1.reshape(1, 32),
                       be1.reshape(1, 32), We2, Wc1[256:288], Wc2,
                       bias)
    partials = _scatter_kernel(x3n.reshape(3 * N), src, dst,
                               c.reshape(E))
    out3n = _reduce(partials.reshape(NW, 3, N), x3n)
    return out3n.T.reshape(B, N, 3)


# speedup vs baseline: 1.5088x; 1.0445x over previous
import functools

import jax
import jax.numpy as jnp
from jax import lax
from jax.experimental import pallas as pl
from jax.experimental.pallas import tpu as pltpu
from jax.experimental.pallas import tpu_sc as plsc

N = 10000
E = 320000
H = 128
HW = H // 2
NC, NS, LANES = 2, 16, 16
NW = NC * NS
EPW = E // NW
GC = 80
NGC = EPW // GC
SC2 = 2000
NSC2 = EPW // SC2
G16 = SC2 // LANES
EH = E // 2

_SC_MESH = plsc.VectorSubcoreMesh(core_axis_name="c", subcore_axis_name="s")


def _pack_bf16_pairs(p):
    b = lax.bitcast_convert_type(p, jnp.int32)
    r = (b + 0x7FFF + ((b >> 16) & 1)) >> 16
    lo = r[:, :HW] & 0xFFFF
    hi = r[:, HW:] << 16
    return hi | lo


def _unpack_bf16_pairs(g):
    f_lo = lax.bitcast_convert_type(g << 16, jnp.float32)
    f_hi = lax.bitcast_convert_type((g >> 16) << 16, jnp.float32)
    return jnp.concatenate([f_lo, f_hi], axis=1)


def _precompute_body(cond_ref, wc1_ref, bc1_ref, be2_ref, t_ref,
                     psp_ref, pdp_ref, bias_ref):
    cnd = cond_ref[...]
    wa = wc1_ref[0:128, :]
    wts = wc1_ref[127:128, :]
    wb = wc1_ref[128:256, :]
    wtd = wc1_ref[255:256, :]
    wc = wc1_ref[256:288, :]
    ps = jnp.dot(cnd, wa, preferred_element_type=jnp.float32, precision=lax.Precision.HIGHEST)
    pd = jnp.dot(cnd, wb, preferred_element_type=jnp.float32, precision=lax.Precision.HIGHEST)
    psp_ref[...] = _pack_bf16_pairs(ps)
    pdp_ref[...] = _pack_bf16_pairs(pd)
    bias_ref[...] = (bc1_ref[...]
                     + jnp.dot(be2_ref[...], wc,
                               preferred_element_type=jnp.float32, precision=lax.Precision.HIGHEST)
                     + t_ref[...] * (wts + wtd))


_precompute = pl.pallas_call(
    _precompute_body,
    out_shape=[jax.ShapeDtypeStruct((N, HW), jnp.int32),
               jax.ShapeDtypeStruct((N, HW), jnp.int32),
               jax.ShapeDtypeStruct((1, H), jnp.float32)],
)


GCH = GC // 2
EPWH = EPW // 2


@functools.partial(
    pl.kernel,
    out_type=jax.ShapeDtypeStruct((EH, H), jnp.int32),
    mesh=_SC_MESH,
    scratch_types=[
        pltpu.VMEM((GC,), jnp.int32), pltpu.VMEM((GC,), jnp.int32),
        pltpu.VMEM((GC,), jnp.int32), pltpu.VMEM((GC,), jnp.int32),
        pltpu.VMEM((GC, HW), jnp.int32), pltpu.VMEM((GC, HW), jnp.int32),
        pltpu.VMEM((GC, HW), jnp.int32), pltpu.VMEM((GC, HW), jnp.int32),
        pltpu.VMEM((GCH, H), jnp.int32), pltpu.VMEM((GCH, H), jnp.int32),
        pltpu.SemaphoreType.DMA, pltpu.SemaphoreType.DMA,
        pltpu.SemaphoreType.DMA, pltpu.SemaphoreType.DMA,
        pltpu.SemaphoreType.DMA, pltpu.SemaphoreType.DMA,
        pltpu.SemaphoreType.DMA, pltpu.SemaphoreType.DMA,
        pltpu.SemaphoreType.DMA, pltpu.SemaphoreType.DMA,
    ],
    compiler_params=pltpu.CompilerParams(use_tc_tiling_on_sc=False,
                                         needs_layout_passes=False),
)
def _gather_kernel(psp_hbm, pdp_hbm, src_hbm, dst_hbm, gsum_hbm,
                   is0, is1, id0, id1, bs0, bs1, bd0, bd1, b20, b21,
                   sis0, sis1, sid0, sid1, sgs0, sgs1, sgd0, sgd1,
                   sw0, sw1):
    wid = lax.axis_index("s") * NC + lax.axis_index("c")
    base = wid * EPW
    baseh = wid * EPWH
    IS, ID = (is0, is1), (id0, id1)
    BS, BD = (bs0, bs1), (bd0, bd1)
    B2 = (b20, b21)
    SIS, SID = (sis0, sis1), (sid0, sid1)
    SGS, SGD = (sgs0, sgs1), (sgd0, sgd1)
    SW = (sw0, sw1)

    def idx_cp(b, c):
        off = base + c * GC
        return (pltpu.make_async_copy(src_hbm.at[pl.ds(off, GC)], IS[b], SIS[b]),
                pltpu.make_async_copy(dst_hbm.at[pl.ds(off, GC)], ID[b], SID[b]))

    def issue_idx(b, c):
        c1, c2 = idx_cp(b, c)
        c1.start()
        c2.start()

    def wait_idx(b):
        c1, c2 = idx_cp(b, 0)
        c1.wait()
        c2.wait()

    def gath(b):
        return (pltpu.make_async_copy(psp_hbm.at[IS[b]], BS[b], SGS[b]),
                pltpu.make_async_copy(pdp_hbm.at[ID[b]], BD[b], SGD[b]))

    def issue_g(b):
        g1, g2 = gath(b)
        g1.start()
        g2.start()

    def wait_g(b):
        g1, g2 = gath(b)
        g1.wait()
        g2.wait()

    def combine(b):
        bs, bd, b2 = BS[b], BD[b], B2[b]

        def row(r, carry):
            for half in (0, 1):
                for k in range(4):
                    s_w = bs[2 * r + half, pl.ds(k * LANES, LANES)]
                    d_w = bd[2 * r + half, pl.ds(k * LANES, LANES)]
                    s_bf = plsc.bitcast(s_w, jnp.bfloat16)
                    d_bf = plsc.bitcast(d_w, jnp.bfloat16)
                    b2[r, pl.ds(half * HW + k * LANES, LANES)] = (
                        plsc.bitcast(s_bf + d_bf, jnp.int32))
            return carry

        lax.fori_loop(0, GCH, row, 0)

    def wr_cp(b, c):
        offh = baseh + c * GCH
        return pltpu.make_async_copy(B2[b], gsum_hbm.at[pl.ds(offh, GCH)],
                                     SW[b])

    issue_idx(0, 0)
    wait_idx(0)
    issue_g(0)
    issue_idx(1, 1)

    def pair(j, carry):
        c = 2 * j + 1
        wait_idx(1)
        issue_g(1)
        wait_g(0)

        @pl.when(j > 0)
        def _():
            wr_cp(0, 0).wait()
            wr_cp(1, 0).wait()

        combine(0)
        wr_cp(0, c - 1).start()
        issue_idx(0, c + 1)
        wait_idx(0)
        issue_g(0)
        wait_g(1)
        combine(1)
        wr_cp(1, c).start()

        @pl.when(j < (NGC // 2) - 1)
        def _():
            issue_idx(1, c + 2)

        return carry

    lax.fori_loop(0, NGC // 2, pair, 0)
    wait_g(0)
    wr_cp(0, 0).wait()
    combine(0)
    wr_cp(0, NGC - 1).start()
    wr_cp(0, 0).wait()
    wr_cp(1, 0).wait()


BEH = 2000


def _edge_body(gsum_ref, d_ref, we1_ref, be1_ref, we2_ref, wc1c_ref,
               wc2_ref, bias_ref, c_ref):
    g = gsum_ref[...]
    glo = lax.bitcast_convert_type(g << 16, jnp.float32)
    ghi = lax.bitcast_convert_type((g >> 16) << 16, jnp.float32)
    wcomb = jnp.dot(we2_ref[...], wc1c_ref[...],
                    preferred_element_type=jnp.float32,
                    precision=lax.Precision.HIGHEST)
    d2 = d_ref[...]
    bias = bias_ref[...]
    cs = []
    for p in (0, 1):
        gsd = jnp.concatenate([glo[:, p * HW:p * HW + HW],
                               ghi[:, p * HW:p * HW + HW]], axis=1)
        a = d2[:, p:p + 1] * we1_ref[...] + be1_ref[...]
        sa = a * jax.nn.sigmoid(a)
        q = jnp.dot(sa, wcomb, preferred_element_type=jnp.float32,
                    precision=lax.Precision.HIGHEST)
        u = gsd + q + bias
        su = u * jax.nn.sigmoid(u)
        cs.append(jnp.dot(su, wc2_ref[...], preferred_element_type=jnp.float32,
                          precision=lax.Precision.HIGHEST))
    c_ref[...] = jnp.concatenate(cs, axis=1)


_edge_epilogue = pl.pallas_call(
    _edge_body,
    grid=(EH // BEH,),
    in_specs=[
        pl.BlockSpec((BEH, H), lambda i: (i, 0)),
        pl.BlockSpec((BEH, 2), lambda i: (i, 0)),
        pl.BlockSpec((1, 32), lambda i: (0, 0)),
        pl.BlockSpec((1, 32), lambda i: (0, 0)),
        pl.BlockSpec((32, 32), lambda i: (0, 0)),
        pl.BlockSpec((32, H), lambda i: (0, 0)),
        pl.BlockSpec((H, 1), lambda i: (0, 0)),
        pl.BlockSpec((1, H), lambda i: (0, 0)),
    ],
    out_specs=pl.BlockSpec((BEH, 2), lambda i: (i, 0)),
    out_shape=jax.ShapeDtypeStruct((EH, 2), jnp.float32),
)


@functools.partial(
    pl.kernel,
    out_type=jax.ShapeDtypeStruct((NW * 3 * N,), jnp.float32),
    mesh=_SC_MESH,
    scratch_types=[
        pltpu.VMEM((N,), jnp.float32),
        pltpu.VMEM((N,), jnp.float32),
        pltpu.VMEM((N,), jnp.float32),
        pltpu.VMEM((N,), jnp.float32),
        pltpu.VMEM((N,), jnp.float32),
        pltpu.VMEM((N,), jnp.float32),
        pltpu.VMEM((SC2,), jnp.int32),
        pltpu.VMEM((SC2,), jnp.int32),
        pltpu.VMEM((SC2,), jnp.float32),
    ],
    compiler_params=pltpu.CompilerParams(needs_layout_passes=False),
)
def _scatter_kernel(xt_hbm, src_hbm, dst_hbm, c_hbm, out_hbm,
                    xv, yv, zv, ax, ay, az, sv, dv, cv):
    wid = lax.axis_index("s") * NC + lax.axis_index("c")
    base = wid * EPW
    pltpu.sync_copy(xt_hbm.at[pl.ds(0, N)], xv)
    pltpu.sync_copy(xt_hbm.at[pl.ds(N, N)], yv)
    pltpu.sync_copy(xt_hbm.at[pl.ds(2 * N, N)], zv)
    zeros = jnp.zeros((LANES,), jnp.float32)

    def zbody(i, carry):
        ax[pl.ds(i * LANES, LANES)] = zeros
        ay[pl.ds(i * LANES, LANES)] = zeros
        az[pl.ds(i * LANES, LANES)] = zeros
        return carry

    lax.fori_loop(0, N // LANES, zbody, 0)

    def chunk(ci, carry):
        off = base + ci * SC2
        pltpu.sync_copy(src_hbm.at[pl.ds(off, SC2)], sv)
        pltpu.sync_copy(dst_hbm.at[pl.ds(off, SC2)], dv)
        pltpu.sync_copy(c_hbm.at[pl.ds(off, SC2)], cv)

        def grp(g, c2):
            s = sv[pl.ds(g * LANES, LANES)]
            dd = dv[pl.ds(g * LANES, LANES)]
            xs = plsc.load_gather(xv, [s])
            xd = plsc.load_gather(xv, [dd])
            ys = plsc.load_gather(yv, [s])
            yd = plsc.load_gather(yv, [dd])
            zs = plsc.load_gather(zv, [s])
            zd = plsc.load_gather(zv, [dd])
            dx = xs - xd
            dy = ys - yd
            dz = zs - zd
            n2 = jnp.maximum(dx * dx + dy * dy + dz * dz,
                             jnp.float32(1e-16))
            ib = plsc.bitcast(n2, jnp.int32)
            yb = jnp.int32(0x5F3759DF) - lax.shift_right_logical(ib, 1)
            yr = plsc.bitcast(yb, jnp.float32)
            yr = yr * (1.5 - 0.5 * n2 * yr * yr)
            yr = yr * (1.5 - 0.5 * n2 * yr * yr)
            yr = yr * (1.5 - 0.5 * n2 * yr * yr)
            cc = cv[pl.ds(g * LANES, LANES)]
            s_c = cc * yr
            plsc.addupdate_scatter(ax, [dd], s_c * dx)
            plsc.addupdate_scatter(ay, [dd], s_c * dy)
            plsc.addupdate_scatter(az, [dd], s_c * dz)
            return c2

        lax.fori_loop(0, G16, grp, 0)
        return carry

    lax.fori_loop(0, NSC2, chunk, 0)
    obase = wid * (3 * N)
    pltpu.sync_copy(ax, out_hbm.at[pl.ds(obase, N)])
    pltpu.sync_copy(ay, out_hbm.at[pl.ds(obase + N, N)])
    pltpu.sync_copy(az, out_hbm.at[pl.ds(obase + 2 * N, N)])


def _reduce_body(part_ref, xt_ref, o_ref):
    o_ref[...] = xt_ref[...] + jnp.sum(part_ref[...], axis=0)


_reduce = pl.pallas_call(
    _reduce_body,
    out_shape=jax.ShapeDtypeStruct((3, N), jnp.float32),
)


def kernel(x_t, cond, t, edge_index, edge_dist, Wn1, bn1, Wn2, bn2,
           Wc1, bc1, Wc2, We1, be1, We2, be2):
    B = x_t.shape[0]
    src = edge_index[0]
    dst = edge_index[1]
    x3n = x_t.reshape(N, 3).T
    tsc = jnp.full((1, 1), t, dtype=jnp.float32)

    cond0 = jnp.concatenate(
        [cond.reshape(N, H - 1), jnp.zeros((N, 1), jnp.float32)], axis=1)
    psp, pdp, bias = _precompute(cond0, Wc1,
                                 bc1.reshape(1, H), be2.reshape(1, 32), tsc)
    gsum = _gather_kernel(psp, pdp, src, dst)
    c = _edge_epilogue(gsum, edge_dist.reshape(EH, 2), We1.reshape(1, 32),
                       be1.reshape(1, 32), We2, Wc1[256:288], Wc2,
                       bias)
    partials = _scatter_kernel(x3n.reshape(3 * N), src, dst,
                               c.reshape(E))
    out3n = _reduce(partials.reshape(NW, 3, N), x3n)
    return out3n.T.reshape(B, N, 3)
